# baseline (speedup 1.0000x reference)
import functools

import numpy as np
import jax
import jax.numpy as jnp
from jax import lax
from jax.experimental import pallas as pl
from jax.experimental.pallas import tpu as pltpu
from jax.experimental.pallas import tpu_sc as plsc

_DIAG = 2
_CH = 512
_JW = 4
_JL = _CH // _JW


def _perm_np(ut_len: int) -> np.ndarray:
    seq = int(np.sqrt(2 * ut_len + 0.25) - 0.5) + _DIAG
    iu0, iu1 = np.triu_indices(seq, k=_DIAG)
    ids = np.zeros((seq, seq), np.int64)
    ids[iu0, iu1] = np.arange(ut_len)
    return ids[seq - 1 - iu1, seq - 1 - iu0].astype(np.int32)


def _chunk_index_table(batch: int, ut_len: int, perm: np.ndarray) -> np.ndarray:
    n_full = ut_len // _CH
    tail = ut_len - n_full * _CH
    cpb = n_full + (1 if tail else 0)
    out = np.zeros((cpb, _CH), np.int32)
    for c in range(cpb):
        n = _CH if c < n_full else tail
        out[c, :n] = perm[c * _CH:c * _CH + n]
        out[c, n:] = out[c, max(n - 1, 0)]
    return np.tile(out, (batch, 1)).reshape(batch * cpb, _JW, _JL)


@functools.lru_cache(maxsize=None)
def _build(batch: int, ut_len: int, head: int):
    n_full = ut_len // _CH
    tail = ut_len - n_full * _CH
    cpb = n_full + (1 if tail else 0)
    n_chunks = batch * cpb

    info = plsc.get_sparse_core_info()
    nw = info.num_cores * info.num_subcores
    steps = -(-n_chunks // nw)
    mesh = plsc.VectorSubcoreMesh(core_axis_name="c", subcore_axis_name="s")

    @functools.partial(
        pl.kernel,
        mesh=mesh,
        out_type=jax.ShapeDtypeStruct((batch, ut_len, head), jnp.float32),
        compiler_params=pltpu.CompilerParams(use_tc_tiling_on_sc=False),
        scratch_types=[
            pltpu.VMEM((_JW, _JL), jnp.int32),
            pltpu.VMEM((_JW, _JL), jnp.int32),
            pltpu.VMEM((_CH, head), jnp.float32),
            pltpu.VMEM((_CH, head), jnp.float32),
            pltpu.SemaphoreType.DMA,
            pltpu.SemaphoreType.DMA,
        ],
    )
    def gather_kernel(x_hbm, idx_hbm, out_hbm, idx0, idx1, buf0, buf1,
                      sem0, sem1):
        wid = lax.axis_index("s") * info.num_cores + lax.axis_index("c")
        idx_v = (idx0, idx1)
        rows_v = (buf0, buf1)
        sems = (sem0, sem1)

        def fire(slot, cid):
            @pl.when(cid < n_chunks)
            def _():
                b = cid // cpb
                pltpu.sync_copy(idx_hbm.at[cid], idx_v[slot])
                for j in range(_JW):
                    pltpu.async_copy(
                        x_hbm.at[b].at[idx_v[slot].at[j]],
                        rows_v[slot].at[pl.ds(j * _JL, _JL)],
                        sems[slot],
                    )

        def drain_write(slot, cid):
            @pl.when(cid < n_chunks)
            def _():
                pltpu.make_async_copy(
                    x_hbm.at[0, pl.ds(0, _CH)], rows_v[slot],
                    sems[slot]).wait()
                b = cid // cpb
                c = cid - b * cpb

                @pl.when(c < n_full)
                def _():
                    pltpu.sync_copy(
                        rows_v[slot],
                        out_hbm.at[b, pl.ds(c * _CH, _CH)])

                if tail:
                    @pl.when(c == n_full)
                    def _():
                        pltpu.sync_copy(
                            rows_v[slot].at[pl.ds(0, tail)],
                            out_hbm.at[b, pl.ds(n_full * _CH, tail)])

        fire(0, wid)

        def body(u, carry):
            c0 = (2 * u) * nw + wid
            fire(1, c0 + nw)
            drain_write(0, c0)
            fire(0, c0 + 2 * nw)
            drain_write(1, c0 + nw)
            return carry

        lax.fori_loop(0, -(-steps // 2), body, 0)

    return gather_kernel


def kernel(x, reverse):
    batch, ut_len, head = x.shape
    perm_idx = _chunk_index_table(batch, ut_len, _perm_np(ut_len))
    ident_idx = _chunk_index_table(
        batch, ut_len, np.arange(ut_len, dtype=np.int32))
    idx = jnp.where(jnp.asarray(reverse) != 0,
                    jnp.asarray(perm_idx), jnp.asarray(ident_idx))
    return _build(batch, ut_len, head)(x, idx)


# speedup vs baseline: 3.9335x; 3.9335x over previous
import functools

import numpy as np
import jax
import jax.numpy as jnp
from jax import lax
from jax.experimental import pallas as pl
from jax.experimental.pallas import tpu as pltpu
from jax.experimental.pallas import tpu_sc as plsc

_DIAG = 2
_CH = 256
_JW = 2
_JL = _CH // _JW


def _perm_np(ut_len: int) -> np.ndarray:
    seq = int(np.sqrt(2 * ut_len + 0.25) - 0.5) + _DIAG
    iu0, iu1 = np.triu_indices(seq, k=_DIAG)
    ids = np.zeros((seq, seq), np.int64)
    ids[iu0, iu1] = np.arange(ut_len)
    return ids[seq - 1 - iu1, seq - 1 - iu0].astype(np.int32)


def _chunk_index_table(batch: int, ut_len: int, perm: np.ndarray) -> np.ndarray:
    n_full = ut_len // _CH
    tail = ut_len - n_full * _CH
    cpb = n_full + (1 if tail else 0)
    out = np.zeros((cpb, _CH), np.int32)
    for c in range(cpb):
        n = _CH if c < n_full else tail
        out[c, :n] = perm[c * _CH:c * _CH + n]
        out[c, n:] = out[c, max(n - 1, 0)]
    return np.tile(out, (batch, 1)).reshape(batch * cpb, _JW, _JL)


@functools.lru_cache(maxsize=None)
def _build(batch: int, ut_len: int, head: int):
    n_full = ut_len // _CH
    tail = ut_len - n_full * _CH
    cpb = n_full + (1 if tail else 0)
    n_chunks = batch * cpb

    info = plsc.get_sparse_core_info()
    nw = info.num_cores * info.num_subcores
    steps = -(-n_chunks // nw)

    mesh = plsc.VectorSubcoreMesh(core_axis_name="c", subcore_axis_name="s")

    @functools.partial(
        pl.kernel,
        mesh=mesh,
        out_type=jax.ShapeDtypeStruct((batch, ut_len, head), jnp.float32),
        compiler_params=pltpu.CompilerParams(use_tc_tiling_on_sc=True),
        scratch_types=[
            pltpu.VMEM((_JW, _JL), jnp.int32),
            pltpu.VMEM((_JW, _JL), jnp.int32),
            pltpu.VMEM((_CH, 2 * head), jnp.float32),
            pltpu.VMEM((_CH, 2 * head), jnp.float32),
            pltpu.VMEM((_CH, head), jnp.float32),
            pltpu.SemaphoreType.DMA,
            pltpu.SemaphoreType.DMA,
        ],
    )
    def gather_kernel(x_hbm, idx_hbm, out_hbm, idx0, idx1, wide0, wide1,
                      nar, sem0, sem1):
        wid = lax.axis_index("s") * info.num_cores + lax.axis_index("c")
        idx_v = (idx0, idx1)
        wide_v = (wide0, wide1)
        sems = (sem0, sem1)

        def fire(slot, cid):
            @pl.when(cid < n_chunks)
            def _():
                b = cid // cpb
                pltpu.sync_copy(idx_hbm.at[cid], idx_v[slot])
                for j in range(_JW):
                    pltpu.async_copy(
                        x_hbm.at[b].at[idx_v[slot].at[j]],
                        wide_v[slot].at[pl.ds(j * _JL, _JL)],
                        sems[slot],
                    )

        def drain_write(slot, cid):
            @pl.when(cid < n_chunks)
            def _():
                pltpu.make_async_copy(
                    x_hbm.at[0, pl.ds(0, _CH)], wide_v[slot],
                    sems[slot]).wait()

                def compact(r4, carry):
                    for r in range(4):
                        for q in range(head // 16):
                            nar[r4 * 4 + r, pl.ds(q * 16, 16)] = (
                                wide_v[slot][r4 * 4 + r, pl.ds(q * 16, 16)])
                    return carry

                lax.fori_loop(0, _CH // 4, compact, 0)

                b = cid // cpb
                c = cid - b * cpb
                base = pl.multiple_of(c * _CH, _CH)

                @pl.when(c < n_full)
                def _():
                    pltpu.sync_copy(
                        nar, out_hbm.at[b, pl.ds(base, _CH)])

                if tail:
                    @pl.when(c == n_full)
                    def _():
                        pltpu.sync_copy(
                            nar.at[pl.ds(0, tail)],
                            out_hbm.at[b, pl.ds(n_full * _CH, tail)])

        fire(0, wid)

        def body(u, carry):
            c0 = (2 * u) * nw + wid
            fire(1, c0 + nw)
            drain_write(0, c0)
            fire(0, c0 + 2 * nw)
            drain_write(1, c0 + nw)
            return carry

        lax.fori_loop(0, -(-steps // 2), body, 0)

    return gather_kernel


def kernel(x, reverse):
    batch, ut_len, head = x.shape
    perm_idx = _chunk_index_table(batch, ut_len, _perm_np(ut_len))
    ident_idx = _chunk_index_table(
        batch, ut_len, np.arange(ut_len, dtype=np.int32))
    idx = jnp.where(jnp.asarray(reverse) != 0,
                    jnp.asarray(perm_idx), jnp.asarray(ident_idx))
    xp = jnp.pad(x, ((0, 0), (0, 0), (0, head)))
    return _build(batch, ut_len, head)(xp, idx)
